# baseline (speedup 1.0000x reference)
import jax
import jax.numpy as jnp
from jax.experimental import pallas as pl
from jax.experimental.pallas import tpu as pltpu

_N_ROWS = 128
_N_COLS = 100000
_CK = 2000
_NC = _N_COLS // _CK
_SK = 400
_NSUB = _CK // _SK

_ROT_A = (13, 15, 26, 6)
_ROT_B = (17, 29, 16, 24)
_KS = (0, 1, 0x1BD11BDB)


def _rotl(x, r):
    return (x << jnp.uint32(r)) | (x >> jnp.uint32(32 - r))


def _threefry_bits(j):
    ks = tuple(jnp.uint32(k) for k in _KS)
    x1 = j + ks[1]
    x0 = x1
    x1 = _rotl(x1, _ROT_A[0]) ^ x0
    for r in _ROT_A[1:]:
        x0 = x0 + x1
        x1 = _rotl(x1, r)
        x1 = x0 ^ x1
    inject = ((ks[1], ks[2], 1), (ks[2], ks[0], 2), (ks[0], ks[1], 3),
              (ks[1], ks[2], 4), (ks[2], ks[0], 5))
    rots = (_ROT_B, _ROT_A, _ROT_B, _ROT_A)
    for (ka, kb, c), rgroup in zip(inject, rots + ((),)):
        x0 = x0 + ka
        x1 = x1 + kb + jnp.uint32(c)
        for r in rgroup:
            x0 = x0 + x1
            x1 = _rotl(x1, r)
            x1 = x0 ^ x1
    return x0 ^ x1


def _z_sub(x, idx, rt):
    bits = _threefry_bits(idx)
    f = jax.lax.bitcast_convert_type(
        (bits >> jnp.uint32(9)) | jnp.uint32(0x3F800000), jnp.float32)
    u = f - jnp.float32(1.0)
    eps = jnp.float32(1e-20)
    g = -jnp.log(-jnp.log(u + eps) + eps)
    return (x + g) * rt


def _in_copy_x(x_hbm, x_buf, in_sems, c, slot):
    return pltpu.make_async_copy(
        x_hbm.at[pl.ds(c * _CK, _CK), :], x_buf.at[slot], in_sems.at[slot])


def _in_copy_z(o_hbm, x_buf, in_sems, c, slot):
    return pltpu.make_async_copy(
        o_hbm.at[pl.ds(c * _CK, _CK), :], x_buf.at[slot], in_sems.at[slot])


def _out_copy(o_hbm, o_buf, out_sems, c, slot):
    return pltpu.make_async_copy(
        o_buf.at[slot], o_hbm.at[pl.ds(c * _CK, _CK), :], out_sems.at[slot])


def _body(x_hbm, t_ref, o_hbm, x_buf, o_buf, m_ref, s_ref, fin_ref,
          in_sems, out_sems):
    i = pl.program_id(0)
    phase2 = i >= _NC
    c = jax.lax.select(phase2, i - _NC, i)
    slot = jax.lax.rem(i, 2)
    nslot = jax.lax.rem(i + 1, 2)
    rt = jnp.float32(1.0) / t_ref[0].astype(jnp.float32)

    @pl.when(i == 0)
    def _():
        _in_copy_x(x_hbm, x_buf, in_sems, 0, slot).start()
        m_ref[...] = jnp.full((8, _N_ROWS), -jnp.inf, jnp.float32)
        s_ref[...] = jnp.zeros((8, _N_ROWS), jnp.float32)

    nxt = i + 1

    @pl.when(nxt < _NC)
    def _():
        _in_copy_x(x_hbm, x_buf, in_sems, nxt, nslot).start()

    @pl.when((nxt >= _NC) & (nxt < 2 * _NC))
    def _():
        _in_copy_z(o_hbm, x_buf, in_sems, nxt - _NC, nslot).start()

    @pl.when(~phase2)
    def _():
        _in_copy_x(x_hbm, x_buf, in_sems, c, slot).wait()

    @pl.when(phase2)
    def _():
        _in_copy_z(o_hbm, x_buf, in_sems, c, slot).wait()

    @pl.when(i >= 2)
    def _():
        c2 = jax.lax.select(i - 2 >= _NC, i - 2 - _NC, i - 2)
        _out_copy(o_hbm, o_buf, out_sems, c2, slot).wait()

    @pl.when(i == _NC)
    def _():
        m_l = m_ref[...]
        s_l = s_ref[...]
        m1 = jnp.max(m_l, axis=0, keepdims=True)
        s1 = jnp.sum(s_l * jnp.exp(m_l - m1), axis=0, keepdims=True)
        fin_ref[0:1, :] = m1
        fin_ref[1:2, :] = jnp.float32(1.0) / s1

    @pl.when(~phase2)
    def _():
        sub = jax.lax.broadcasted_iota(jnp.uint32, (_SK, _N_ROWS), 0)
        lane = jax.lax.broadcasted_iota(jnp.uint32, (_SK, _N_ROWS), 1)
        base = (c * _CK).astype(jnp.uint32)
        for k in range(_NSUB):
            a0 = k * _SK
            idx = lane * jnp.uint32(_N_COLS) + (base + jnp.uint32(a0)) + sub
            z = _z_sub(x_buf[slot, pl.ds(a0, _SK), :], idx, rt)
            o_buf[slot, pl.ds(a0, _SK), :] = z
            zv = z.reshape(_SK // 8, 8, _N_ROWS)
            m_old = m_ref[...]
            m_new = jnp.maximum(m_old, jnp.max(zv, axis=0))
            ce = jnp.sum(jnp.exp(zv - m_new[None, :, :]), axis=0)
            s_ref[...] = s_ref[...] * jnp.exp(m_old - m_new) + ce
            m_ref[...] = m_new

    @pl.when(phase2)
    def _():
        m1 = fin_ref[0:1, :]
        rs = fin_ref[1:2, :]
        for k in range(_NSUB):
            a0 = k * _SK
            z = x_buf[slot, pl.ds(a0, _SK), :]
            o_buf[slot, pl.ds(a0, _SK), :] = jnp.exp(z - m1) * rs

    _out_copy(o_hbm, o_buf, out_sems, c, slot).start()

    @pl.when(i == 2 * _NC - 1)
    def _():
        c_prev = jax.lax.select(i - 1 >= _NC, i - 1 - _NC, i - 1)
        _out_copy(o_hbm, o_buf, out_sems, c_prev, nslot).wait()
        _out_copy(o_hbm, o_buf, out_sems, c, slot).wait()


def kernel(logits, temperature, use_gpu):
    del use_gpu
    xt = logits.T
    t = jnp.reshape(temperature, (1,))
    yt = pl.pallas_call(
        _body,
        grid=(2 * _NC,),
        in_specs=[
            pl.BlockSpec(memory_space=pl.ANY),
            pl.BlockSpec(memory_space=pltpu.SMEM),
        ],
        out_specs=pl.BlockSpec(memory_space=pl.ANY),
        out_shape=jax.ShapeDtypeStruct((_N_COLS, _N_ROWS), jnp.float32),
        scratch_shapes=[
            pltpu.VMEM((2, _CK, _N_ROWS), jnp.float32),
            pltpu.VMEM((2, _CK, _N_ROWS), jnp.float32),
            pltpu.VMEM((8, _N_ROWS), jnp.float32),
            pltpu.VMEM((8, _N_ROWS), jnp.float32),
            pltpu.VMEM((2, _N_ROWS), jnp.float32),
            pltpu.SemaphoreType.DMA((2,)),
            pltpu.SemaphoreType.DMA((2,)),
        ],
    )(xt, t)
    return yt.T
